# baseline (speedup 1.0000x reference)
import functools

import jax
import jax.numpy as jnp
from jax import lax
from jax.experimental import pallas as pl
from jax.experimental.pallas import tpu as pltpu
from jax.experimental.pallas import tpu_sc as plsc

_OUT = 7
_NB = _OUT * _OUT
_SCALE = 0.25
_NC, _NS, _L = 2, 16, 16
_NW = _NC * _NS


def _tc_transpose(x, h_t):
    B, C, H, W = x.shape

    def body(in_ref, out_ref):
        for y in range(h_t):
            x = in_ref[0, :, y, :]
            u = lax.bitcast_convert_type(x, jnp.int32) + jnp.int32(0x8000)
            u = u.reshape(C // 32, 2, 16, W)
            lo = lax.shift_right_logical(u[:, 0], 16)
            hi = u[:, 1] & jnp.int32(-65536)
            word = (hi | lo).reshape(C // 2, W)
            out_ref[0, pl.ds(y * W, W), :] = jnp.swapaxes(word, 0, 1)

    return pl.pallas_call(
        body,
        grid=(B, H // h_t),
        in_specs=[pl.BlockSpec((1, C, h_t, W), lambda b, i: (b, 0, i, 0))],
        out_specs=pl.BlockSpec((1, h_t * W, C // 2), lambda b, i: (b, i, 0)),
        out_shape=jax.ShapeDtypeStruct((B, H * W, C // 2), jnp.int32),
    )(x)


def _sc_roi_align(featT, roisp, N, C, H, W):
    HW = H * W
    CB = C // _L
    mesh = plsc.VectorSubcoreMesh(
        core_axis_name="c", subcore_axis_name="s",
        num_cores=_NC, num_subcores=_NS)

    GB = _OUT
    NG = _OUT
    GR = GB * _L

    @functools.partial(
        pl.kernel,
        out_type=jax.ShapeDtypeStruct((N * _NB * C,), jnp.float32),
        mesh=mesh,
        scratch_types=[
            pltpu.VMEM((N * 16,), jnp.float32),
            pltpu.VMEM((4 * GR,), jnp.int32),
            pltpu.VMEM((4 * GR,), jnp.float32),
            pltpu.VMEM((4 * GR, C // 2), jnp.int32),
            pltpu.VMEM((2 * _NB * C,), jnp.float32),
            pltpu.SemaphoreType.DMA,
            pltpu.SemaphoreType.DMA((4,)),
            pltpu.SemaphoreType.DMA((2,)),
        ],
    )
    def body(feat_hbm, rois_hbm, out_hbm, rois_v, idx_v, w_v, rows_v, out_v,
             sem_r, sem_g, sem_o):
        wid = lax.axis_index("s") * _NC + lax.axis_index("c")
        iota = lax.iota(jnp.int32, _L)
        oyf = ((iota >> 3) & 1).astype(jnp.float32)
        oxf = ((iota >> 2) & 1).astype(jnp.float32)
        cy = ((iota >> 1) & 1) == 1
        cx = (iota & 1) == 1
        zeros = jnp.zeros((_L,), jnp.float32)

        nr = (N - wid + _NW - 1) // _NW
        pltpu.async_copy(rois_hbm, rois_v, sem_r).wait()

        RC = _NB * C

        def roi_body(ri, carry):
            r = wid + ri * _NW
            q = ri & 1
            oo = q * RC

            @pl.when(ri >= 2)
            def _():
                pltpu.make_async_copy(
                    out_v.at[pl.ds(oo, RC)],
                    out_hbm.at[pl.ds(0, RC)],
                    sem_o.at[q]).wait()

            roi16 = rois_v[pl.ds(r * 16, _L)]
            bidx = roi16[0].astype(jnp.int32)
            x1 = roi16[1] * _SCALE
            y1 = roi16[2] * _SCALE
            x2 = roi16[3] * _SCALE
            y2 = roi16[4] * _SCALE
            bin_w = jnp.maximum(x2 - x1, 1.0) * (1.0 / _OUT)
            bin_h = jnp.maximum(y2 - y1, 1.0) * (1.0 / _OUT)
            base = bidx * HW

            def fire_group(g):
                p = g & 3
                po = p * GR
                byf = g.astype(jnp.float32)
                Y = y1 + (byf + (oyf + 0.5) * 0.5) * bin_h
                valid_y = (Y >= -1.0) & (Y <= float(H))
                y = jnp.maximum(Y, 0.0)
                ylo = y.astype(jnp.int32)
                yc = ylo >= H - 1
                y_low = jnp.minimum(ylo, H - 1)
                y_high = jnp.minimum(ylo + 1, H - 1)
                ly = jnp.where(yc, zeros, y - y_low.astype(jnp.float32))
                wy = jnp.where(cy, ly, 1.0 - ly)
                rowi = jnp.where(cy, y_high, y_low)
                for t in range(GB):
                    bxf = float(t)
                    X = x1 + (bxf + (oxf + 0.5) * 0.5) * bin_w
                    valid = valid_y & (X >= -1.0) & (X <= float(W))
                    x = jnp.maximum(X, 0.0)
                    xlo = x.astype(jnp.int32)
                    xc = xlo >= W - 1
                    x_low = jnp.minimum(xlo, W - 1)
                    x_high = jnp.minimum(xlo + 1, W - 1)
                    lx = jnp.where(xc, zeros, x - x_low.astype(jnp.float32))
                    wx = jnp.where(cx, lx, 1.0 - lx)
                    w = wy * wx * jnp.where(valid, jnp.full((_L,), 0.25),
                                            zeros)
                    coli = jnp.where(cx, x_high, x_low)
                    idx = base + rowi * W + coli
                    idx_v[pl.ds(po + t * _L, _L)] = idx
                    w_v[pl.ds(po + t * _L, _L)] = w
                pltpu.async_copy(
                    feat_hbm.at[idx_v.at[pl.ds(po, GR)]],
                    rows_v.at[pl.ds(po, GR)],
                    sem_g.at[p])

            def drain_group(g):
                p = g & 3
                po = p * GR
                pltpu.make_async_copy(
                    feat_hbm.at[pl.ds(0, GR)],
                    rows_v.at[pl.ds(po, GR)],
                    sem_g.at[p]).wait()
                b0 = g * GB
                for t in range(GB):
                    wv = w_v[pl.ds(po + t * _L, _L)]
                    accs = [zeros] * CB
                    for k in range(_L):
                        wk = wv[k]
                        for j in range(CB // 2):
                            v = rows_v[po + t * _L + k,
                                       pl.ds(j * _L, _L)]
                            lo = lax.bitcast_convert_type(
                                v << 16, jnp.float32)
                            hi = lax.bitcast_convert_type(
                                v & jnp.int32(-65536), jnp.float32)
                            accs[2 * j] = accs[2 * j] + wk * lo
                            accs[2 * j + 1] = accs[2 * j + 1] + wk * hi
                    ob = oo + (b0 + t) * C
                    for j in range(CB):
                        out_v[pl.ds(ob + j * _L, _L)] = accs[j]

            def grp_body(g, carry2):
                @pl.when(g < NG)
                def _():
                    fire_group(g)

                @pl.when(g >= 2)
                def _():
                    drain_group(g - 2)

                return carry2

            lax.fori_loop(0, NG + 2, grp_body, 0)
            dst = out_hbm.at[pl.ds(pl.multiple_of(r * RC, 8), RC)]
            pltpu.async_copy(out_v.at[pl.ds(oo, RC)], dst, sem_o.at[q])
            return carry

        lax.fori_loop(0, nr, roi_body, 0)
        for q in range(2):
            pltpu.make_async_copy(
                out_v.at[pl.ds(q * RC, RC)],
                out_hbm.at[pl.ds(0, RC)],
                sem_o.at[q]).wait()

    return body(featT, roisp)


def _tc_out_transpose(x, n, nb, c, rt):
    def body(in_ref, out_ref):
        out_ref[...] = jnp.swapaxes(in_ref[...], 1, 2)

    return pl.pallas_call(
        body,
        grid=(n // rt,),
        in_specs=[pl.BlockSpec((rt, nb, c), lambda i: (i, 0, 0))],
        out_specs=pl.BlockSpec((rt, c, nb), lambda i: (i, 0, 0)),
        out_shape=jax.ShapeDtypeStruct((n, c, nb), x.dtype),
    )(x.reshape(n, nb, c))


def kernel(features, rois):
    B, C, H, W = features.shape
    N = rois.shape[0]
    featT = _tc_transpose(features, h_t=8)
    featT = featT.reshape(B * H * W, C // 2)
    roisp = jnp.concatenate(
        [rois, jnp.zeros((N, 16 - rois.shape[1]), rois.dtype)],
        axis=1).reshape(-1)
    out = _sc_roi_align(featT, roisp, N, C, H, W)
    out = _tc_out_transpose(out, N, _NB, C, rt=25)
    return out.reshape(N, C, _OUT, _OUT)


# speedup vs baseline: 1.2065x; 1.2065x over previous
import functools

import jax
import jax.numpy as jnp
from jax import lax
from jax.experimental import pallas as pl
from jax.experimental.pallas import tpu as pltpu
from jax.experimental.pallas import tpu_sc as plsc

_OUT = 7
_NB = _OUT * _OUT
_SCALE = 0.25
_NC, _NS, _L = 2, 16, 16
_NW = _NC * _NS


def _tc_transpose(x, hw_t):
    B, C, HW = x.shape

    def body(in_ref, out_ref):
        x = in_ref[...].reshape(C // 32, 2, 16, hw_t)
        u = lax.bitcast_convert_type(x, jnp.int32) + jnp.int32(0x8000)
        lo = lax.shift_right_logical(u[:, 0], 16)
        hi = u[:, 1] & jnp.int32(-65536)
        word = hi | lo
        out_ref[...] = jnp.swapaxes(word.reshape(1, C // 2, hw_t), 1, 2)

    return pl.pallas_call(
        body,
        grid=(B, HW // hw_t),
        in_specs=[pl.BlockSpec((1, C, hw_t), lambda b, i: (b, 0, i))],
        out_specs=pl.BlockSpec((1, hw_t, C // 2), lambda b, i: (b, i, 0)),
        out_shape=jax.ShapeDtypeStruct((B, HW, C // 2), jnp.int32),
    )(x)


def _sc_roi_align(featT, roisp, N, C, H, W):
    HW = H * W
    CB = C // _L
    mesh = plsc.VectorSubcoreMesh(
        core_axis_name="c", subcore_axis_name="s",
        num_cores=_NC, num_subcores=_NS)

    GB = _OUT
    NG = _OUT
    GR = GB * _L

    @functools.partial(
        pl.kernel,
        out_type=jax.ShapeDtypeStruct((N * _NB * C,), jnp.float32),
        mesh=mesh,
        scratch_types=[
            pltpu.VMEM((N * 16,), jnp.float32),
            pltpu.VMEM((4 * GR,), jnp.int32),
            pltpu.VMEM((4 * GR,), jnp.float32),
            pltpu.VMEM((4 * GR, C // 2), jnp.int32),
            pltpu.VMEM((2 * _NB * C,), jnp.float32),
            pltpu.SemaphoreType.DMA,
            pltpu.SemaphoreType.DMA((4,)),
            pltpu.SemaphoreType.DMA((2,)),
        ],
    )
    def body(feat_hbm, rois_hbm, out_hbm, rois_v, idx_v, w_v, rows_v, out_v,
             sem_r, sem_g, sem_o):
        wid = lax.axis_index("s") * _NC + lax.axis_index("c")
        iota = lax.iota(jnp.int32, _L)
        oyf = ((iota >> 3) & 1).astype(jnp.float32)
        oxf = ((iota >> 2) & 1).astype(jnp.float32)
        cy = ((iota >> 1) & 1) == 1
        cx = (iota & 1) == 1
        zeros = jnp.zeros((_L,), jnp.float32)

        nr = (N - wid + _NW - 1) // _NW
        pltpu.async_copy(rois_hbm, rois_v, sem_r).wait()

        RC = _NB * C

        def roi_body(ri, carry):
            r = wid + ri * _NW
            q = ri & 1
            oo = q * RC

            @pl.when(ri >= 2)
            def _():
                pltpu.make_async_copy(
                    out_v.at[pl.ds(oo, RC)],
                    out_hbm.at[pl.ds(0, RC)],
                    sem_o.at[q]).wait()

            roi16 = rois_v[pl.ds(r * 16, _L)]
            bidx = roi16[0].astype(jnp.int32)
            x1 = roi16[1] * _SCALE
            y1 = roi16[2] * _SCALE
            x2 = roi16[3] * _SCALE
            y2 = roi16[4] * _SCALE
            bin_w = jnp.maximum(x2 - x1, 1.0) * (1.0 / _OUT)
            bin_h = jnp.maximum(y2 - y1, 1.0) * (1.0 / _OUT)
            base = bidx * HW

            def fire_group(g):
                p = g & 3
                po = p * GR
                byf = g.astype(jnp.float32)
                Y = y1 + (byf + (oyf + 0.5) * 0.5) * bin_h
                valid_y = (Y >= -1.0) & (Y <= float(H))
                y = jnp.maximum(Y, 0.0)
                ylo = y.astype(jnp.int32)
                yc = ylo >= H - 1
                y_low = jnp.minimum(ylo, H - 1)
                y_high = jnp.minimum(ylo + 1, H - 1)
                ly = jnp.where(yc, zeros, y - y_low.astype(jnp.float32))
                wy = jnp.where(cy, ly, 1.0 - ly)
                rowi = jnp.where(cy, y_high, y_low)
                for t in range(GB):
                    bxf = float(t)
                    X = x1 + (bxf + (oxf + 0.5) * 0.5) * bin_w
                    valid = valid_y & (X >= -1.0) & (X <= float(W))
                    x = jnp.maximum(X, 0.0)
                    xlo = x.astype(jnp.int32)
                    xc = xlo >= W - 1
                    x_low = jnp.minimum(xlo, W - 1)
                    x_high = jnp.minimum(xlo + 1, W - 1)
                    lx = jnp.where(xc, zeros, x - x_low.astype(jnp.float32))
                    wx = jnp.where(cx, lx, 1.0 - lx)
                    w = wy * wx * jnp.where(valid, jnp.full((_L,), 0.25),
                                            zeros)
                    coli = jnp.where(cx, x_high, x_low)
                    idx = base + rowi * W + coli
                    idx_v[pl.ds(po + t * _L, _L)] = idx
                    w_v[pl.ds(po + t * _L, _L)] = w
                pltpu.async_copy(
                    feat_hbm.at[idx_v.at[pl.ds(po, GR)]],
                    rows_v.at[pl.ds(po, GR)],
                    sem_g.at[p])

            def drain_group(g):
                p = g & 3
                po = p * GR
                pltpu.make_async_copy(
                    feat_hbm.at[pl.ds(0, GR)],
                    rows_v.at[pl.ds(po, GR)],
                    sem_g.at[p]).wait()
                b0 = g * GB
                for t in range(GB):
                    wv = w_v[pl.ds(po + t * _L, _L)]
                    accs = [zeros] * CB
                    for k in range(_L):
                        wk = wv[k]
                        for j in range(CB // 2):
                            v = rows_v[po + t * _L + k,
                                       pl.ds(j * _L, _L)]
                            lo = lax.bitcast_convert_type(
                                v << 16, jnp.float32)
                            hi = lax.bitcast_convert_type(v, jnp.float32)
                            accs[2 * j] = accs[2 * j] + wk * lo
                            accs[2 * j + 1] = accs[2 * j + 1] + wk * hi
                    ob = oo + (b0 + t) * C
                    for j in range(CB):
                        out_v[pl.ds(ob + j * _L, _L)] = accs[j]

            def grp_body(g, carry2):
                @pl.when(g < NG)
                def _():
                    fire_group(g)

                @pl.when(g >= 2)
                def _():
                    drain_group(g - 2)

                return carry2

            lax.fori_loop(0, NG + 2, grp_body, 0)
            dst = out_hbm.at[pl.ds(pl.multiple_of(r * RC, 8), RC)]
            pltpu.async_copy(out_v.at[pl.ds(oo, RC)], dst, sem_o.at[q])
            return carry

        lax.fori_loop(0, nr, roi_body, 0)
        for q in range(2):
            pltpu.make_async_copy(
                out_v.at[pl.ds(q * RC, RC)],
                out_hbm.at[pl.ds(0, RC)],
                sem_o.at[q]).wait()

    return body(featT, roisp)


def _tc_out_transpose(x, n, nb, c, rt):
    def body(in_ref, out_ref):
        out_ref[...] = jnp.swapaxes(in_ref[...], 1, 2)

    return pl.pallas_call(
        body,
        grid=(n // rt,),
        in_specs=[pl.BlockSpec((rt, nb, c), lambda i: (i, 0, 0))],
        out_specs=pl.BlockSpec((rt, c, nb), lambda i: (i, 0, 0)),
        out_shape=jax.ShapeDtypeStruct((n, c, nb), x.dtype),
    )(x.reshape(n, nb, c))


def kernel(features, rois):
    B, C, H, W = features.shape
    N = rois.shape[0]
    featT = _tc_transpose(features.reshape(B, C, H * W), hw_t=3200)
    featT = featT.reshape(B * H * W, C // 2)
    roisp = jnp.concatenate(
        [rois, jnp.zeros((N, 16 - rois.shape[1]), rois.dtype)],
        axis=1).reshape(-1)
    out = _sc_roi_align(featT, roisp, N, C, H, W)
    out = _tc_out_transpose(out, N, _NB, C, rt=25)
    return out.reshape(N, C, _OUT, _OUT)
